# baseline (speedup 1.0000x reference)
import functools

import jax
import jax.numpy as jnp
from jax import lax
from jax.experimental import pallas as pl
from jax.experimental.pallas import tpu as pltpu
from jax.experimental.pallas import tpu_sc as plsc

D_MODEL = 1024
LANES = 16


@functools.lru_cache(maxsize=None)
def _build_sc_kernel(n_rows: int, d: int, vocab: int):
    info = plsc.get_sparse_core_info()
    nc, ns = info.num_cores, info.num_subcores
    nw = nc * ns
    assert n_rows % nw == 0
    rows_per_w = n_rows // nw
    chunk = 16
    steps = rows_per_w // chunk
    nbuf = 3
    vecs_per_row = d // LANES

    mesh = plsc.VectorSubcoreMesh(core_axis_name="c", subcore_axis_name="s")

    @functools.partial(
        pl.kernel,
        mesh=mesh,
        out_type=jax.ShapeDtypeStruct((n_rows, d), jnp.float32),
        scratch_types=[
            pltpu.VMEM((steps, chunk), jnp.int32),
            pltpu.VMEM((nbuf, chunk, d), jnp.float32),
            pltpu.VMEM((nbuf, chunk, d), jnp.float32),
            pltpu.SemaphoreType.DMA,
            pltpu.SemaphoreType.DMA,
            pltpu.SemaphoreType.DMA,
        ],
    )
    def k(x_hbm, idx_hbm, pe_hbm, out_hbm, idx_v, x_v, pe_v, sem0, sem1, sem2):
        sems = (sem0, sem1, sem2)
        wid = lax.axis_index("s") * nc + lax.axis_index("c")
        base = wid * rows_per_w

        pltpu.sync_copy(idx_hbm.at[pl.ds(wid * steps, steps)], idx_v)

        in_flight = [None] * steps

        def stage_in(s):
            b = s % nbuf
            r0 = base + s * chunk
            cx = pltpu.async_copy(
                x_hbm.at[pl.ds(r0, chunk)], x_v.at[b], sems[b])
            cg = pltpu.async_copy(
                pe_hbm.at[idx_v.at[s]], pe_v.at[b], sems[b])
            in_flight[s] = (cx, cg)

        def add_and_store(s):
            b = s % nbuf
            r0 = base + s * chunk
            cx, cg = in_flight[s]
            cx.wait()
            cg.wait()

            def add_row(r, carry):
                for c in range(vecs_per_row):
                    sl = pl.ds(c * LANES, LANES)
                    plsc.addupdate(x_v.at[b, r, sl], pe_v[b, r, sl])
                return carry

            lax.fori_loop(0, chunk, add_row, 0, unroll=False)
            in_flight[s] = pltpu.async_copy(
                x_v.at[b], out_hbm.at[pl.ds(r0, chunk)], sems[b])

        def drain(s):
            in_flight[s].wait()

        for s in range(steps + 2):
            if s >= 2:
                drain(s - 2)
            if s < steps:
                stage_in(s)
            if 1 <= s <= steps:
                add_and_store(s - 1)

    return k


def kernel(x, rel_times, pe_table):
    b, t, d = x.shape
    n = b * t
    xf = x.reshape(n, d)
    idx = rel_times.reshape(n // 16, 16).astype(jnp.int32)
    out = _build_sc_kernel(n, d, pe_table.shape[0])(xf, idx, pe_table)
    return out.reshape(b, t, d)


# speedup vs baseline: 1.2741x; 1.2741x over previous
import functools

import jax
import jax.numpy as jnp
from jax import lax
from jax.experimental import pallas as pl
from jax.experimental.pallas import tpu as pltpu
from jax.experimental.pallas import tpu_sc as plsc

D_MODEL = 1024
LANES = 16


@functools.lru_cache(maxsize=None)
def _build_sc_kernel(n_rows: int, d: int, vocab: int):
    info = plsc.get_sparse_core_info()
    nc, ns = info.num_cores, info.num_subcores
    nw = nc * ns
    assert n_rows % nw == 0
    rows_per_w = n_rows // nw
    chunk = 16
    steps = rows_per_w // chunk
    nbuf = 3
    vecs_per_row = d // LANES

    mesh = plsc.VectorSubcoreMesh(core_axis_name="c", subcore_axis_name="s")

    @functools.partial(
        pl.kernel,
        mesh=mesh,
        out_type=jax.ShapeDtypeStruct((n_rows, d), jnp.float32),
        scratch_types=[
            pltpu.VMEM((rows_per_w,), jnp.int32),
            pltpu.VMEM((nbuf, chunk, d), jnp.float32),
            pltpu.VMEM((nbuf, chunk, d), jnp.float32),
            pltpu.SemaphoreType.DMA,
            pltpu.SemaphoreType.DMA,
            pltpu.SemaphoreType.DMA,
        ],
    )
    def k(x_hbm, idx_hbm, pe_hbm, out_hbm, idx_v, x_v, pe_v, sem0, sem1, sem2):
        sems = (sem0, sem1, sem2)
        wid = lax.axis_index("s") * nc + lax.axis_index("c")
        base = wid * rows_per_w

        pltpu.sync_copy(idx_hbm.at[pl.ds(base, rows_per_w)], idx_v)

        in_flight = [None] * steps

        def stage_in(s):
            b = s % nbuf
            r0 = base + s * chunk
            cx = pltpu.async_copy(
                x_hbm.at[pl.ds(r0, chunk)], x_v.at[b], sems[b])
            cg = pltpu.async_copy(
                pe_hbm.at[idx_v.at[pl.ds(s * chunk, chunk)]], pe_v.at[b],
                sems[b])
            in_flight[s] = (cx, cg)

        def add_and_store(s):
            b = s % nbuf
            r0 = base + s * chunk
            cx, cg = in_flight[s]
            cx.wait()
            cg.wait()

            def add_row(r, carry):
                for c in range(vecs_per_row):
                    sl = pl.ds(c * LANES, LANES)
                    plsc.addupdate(x_v.at[b, r, sl], pe_v[b, r, sl])
                return carry

            lax.fori_loop(0, chunk, add_row, 0, unroll=False)
            in_flight[s] = pltpu.async_copy(
                x_v.at[b], out_hbm.at[pl.ds(r0, chunk)], sems[b])

        def drain(s):
            in_flight[s].wait()

        for s in range(steps + 2):
            if s >= 2:
                drain(s - 2)
            if s < steps:
                stage_in(s)
            if 1 <= s <= steps:
                add_and_store(s - 1)

    return k


def kernel(x, rel_times, pe_table):
    b, t, d = x.shape
    n = b * t
    xf = x.reshape(n, d)
    idx = rel_times.reshape(n).astype(jnp.int32)
    out = _build_sc_kernel(n, d, pe_table.shape[0])(xf, idx, pe_table)
    return out.reshape(b, t, d)
